# baseline (speedup 1.0000x reference)
import jax
import jax.numpy as jnp
from jax.experimental import pallas as pl
from jax.experimental.pallas import tpu as pltpu

_G = 76
_N = _G * _G
_STRIDE = 8.0
_AW = (116.0, 156.0, 373.0)
_AH = (90.0, 198.0, 326.0)

_K = 8
_CHUNKS = [(k * 720, 720) for k in range(7)] + [(5040, 736)]


def _decode_kernel(x_ref, o_ref, scratch, sems):
    i = pl.program_id(0)
    n_steps = pl.num_programs(0)
    slot = i % 2

    def _store_copies(step, slot_):
        for k, (start, size) in enumerate(_CHUNKS):
            yield pltpu.make_async_copy(
                scratch.at[slot_, pl.ds(start, size), :],
                o_ref.at[step, pl.ds(start, size), :],
                sems.at[slot_, k],
            )

    @pl.when(i >= 2)
    def _():
        for c in _store_copies(i - 2, slot):
            c.wait()

    a = i % 3
    ch = x_ref[0]

    col = jax.lax.broadcasted_iota(jnp.int32, (1, _N), 1)
    xoff = (col % _G).astype(jnp.float32)
    yoff = (col // _G).astype(jnp.float32)

    sxy = jax.nn.sigmoid(ch[0:2])
    bx = (sxy[0:1] + xoff) * _STRIDE
    by = (sxy[1:2] + yoff) * _STRIDE

    aw = jnp.where(a == 0, _AW[0], jnp.where(a == 1, _AW[1], _AW[2]))
    ah = jnp.where(a == 0, _AH[0], jnp.where(a == 1, _AH[1], _AH[2]))
    ewh = jnp.exp(ch[2:4])
    bw = ewh[0:1] * aw
    bh = ewh[1:2] * ah

    rest = jax.nn.sigmoid(ch[4:85])

    full = jnp.concatenate([bx, by, bw, bh, rest], axis=0)
    scratch[slot] = full.T

    for c in _store_copies(i, slot):
        c.start()

    @pl.when(i == n_steps - 1)
    def _():
        for c in _store_copies(i - 1, (i - 1) % 2):
            c.wait()
        for c in _store_copies(i, slot):
            c.wait()


def kernel(x):
    b = x.shape[0]
    xr = x.reshape(b * 3, 85, _N)
    out = pl.pallas_call(
        _decode_kernel,
        grid=(b * 3,),
        in_specs=[pl.BlockSpec((1, 85, _N), lambda i: (i, 0, 0))],
        out_specs=pl.BlockSpec(memory_space=pltpu.MemorySpace.HBM),
        out_shape=jax.ShapeDtypeStruct((b * 3, _N, 85), jnp.float32),
        scratch_shapes=[
            pltpu.VMEM((2, _N, 85), jnp.float32),
            pltpu.SemaphoreType.DMA((2, _K)),
        ],
    )(xr)
    return (out.reshape(b, 3 * _N, 85), 0)
